# Initial kernel scaffold; baseline (speedup 1.0000x reference)
import jax
import jax.numpy as jnp
from jax.experimental import pallas as pl


def kernel(L0_idx, L0_val, L1_idx, L1_val, L2_idx, L2_val, D0, D1, D2, adD0, adD1, adD2, x0, x1, x2, t0_1, b0_1, t0_2, b0_2, t0_3, b0_3, t1_1, b1_1, t1_2, b1_2, t1_3, b1_3, t2_1, b2_1, t2_2, b2_2, t2_3, b2_3):
    raise NotImplementedError("write your pallas kernel here")


# speedup vs baseline: 4.3742x; 4.3742x over previous
import functools

import jax
import jax.numpy as jnp
from jax import lax
from jax.experimental import pallas as pl
from jax.experimental.pallas import tpu as pltpu
from jax.experimental.pallas import tpu_sc as plsc

F32 = jnp.float32
I32 = jnp.int32
NS = 16
G = 128
BM = 1024
KTERMS = 5


def _zchunk(rng, cap=512):
    best = 8
    for d in range(8, cap + 1, 8):
        if rng % d == 0:
            best = d
    return best


def _make_cheb(M, NNZp):
    rng = M // NS
    nper = NNZp // NS
    nch = nper // G
    CH = _zchunk(rng)
    mesh = plsc.VectorSubcoreMesh(core_axis_name="c", subcore_axis_name="s")

    @functools.partial(
        pl.kernel,
        out_type=jax.ShapeDtypeStruct((4, 2, M, 16), F32),
        mesh=mesh,
        compiler_params=pltpu.CompilerParams(use_tc_tiling_on_sc=False),
        scratch_types=[
            pltpu.VMEM_SHARED((M, 16), F32),
            pltpu.VMEM_SHARED((M, 16), F32),
            pltpu.VMEM((CH, 16), F32),
            pltpu.VMEM((CH, 16), F32),
            pltpu.VMEM((G, 16), F32),
            pltpu.VMEM((G,), I32),
            pltpu.VMEM((G,), I32),
            pltpu.VMEM((G,), F32),
            pltpu.SemaphoreType.DMA,
        ],
    )
    def cheb(xin, rows, cols, val, tout,
             Xsp, Ysh, cbuf, pbuf, rowbuf, colb, rowb, valb, sem):
        c = lax.axis_index("c")
        s = lax.axis_index("s")
        base = s * rng
        nbase = s * nper
        nq = rng // CH

        pltpu.sync_copy(xin.at[c, pl.ds(base, rng)], Xsp.at[pl.ds(base, rng)])

        for kk in (1, 2, 3, 4):
            def zfill(i, _):
                cbuf[i, :] = jnp.zeros((16,), F32)
                return 0
            lax.fori_loop(0, CH, zfill, 0)

            def zrow(i, _):
                pltpu.sync_copy(cbuf, Ysh.at[pl.ds(base + i * CH, CH)])
                return 0
            lax.fori_loop(0, nq, zrow, 0)
            plsc.subcore_barrier()

            def chunk(i, _):
                off = nbase + i * G
                pltpu.sync_copy(cols.at[pl.ds(off, G)], colb)
                pltpu.sync_copy(rows.at[pl.ds(off, G)], rowb)
                pltpu.sync_copy(val.at[pl.ds(off, G)], valb)
                pltpu.async_copy(Xsp.at[colb], rowbuf, sem).wait()

                dnums = lax.GatherDimensionNumbers(
                    offset_dims=(), collapsed_slice_dims=(0,),
                    start_index_map=(0,))

                def scale16(j, _):
                    v16 = valb[pl.ds(j * 16, 16)]
                    for l in range(16):
                        b = lax.gather(
                            v16, jnp.full((16, 1), l, I32), dnums, (1,),
                            mode=lax.GatherScatterMode.PROMISE_IN_BOUNDS)
                        jj = j * 16 + l
                        rowbuf[jj, :] = rowbuf[jj, :] * b
                    return 0
                lax.fori_loop(0, G // 16, scale16, 0)
                pltpu.sync_copy(rowbuf, Ysh.at[rowb], add=True)
                return 0
            lax.fori_loop(0, nch, chunk, 0)
            plsc.subcore_barrier()

            for q_static in (None,):
                def wout(q, _):
                    off = base + q * CH
                    pltpu.sync_copy(Ysh.at[pl.ds(off, CH)], cbuf)
                    if kk > 1:
                        if kk == 2:
                            pltpu.sync_copy(xin.at[c, pl.ds(off, CH)], pbuf)
                        else:
                            pltpu.sync_copy(
                                tout.at[kk - 3, c, pl.ds(off, CH)], pbuf)

                        def comb(i, _):
                            cbuf[i, :] = 2.0 * cbuf[i, :] - pbuf[i, :]
                            return 0
                        lax.fori_loop(0, CH, comb, 0)
                    pltpu.sync_copy(cbuf, tout.at[kk - 1, c, pl.ds(off, CH)])
                    if kk < 4:
                        pltpu.sync_copy(cbuf, Xsp.at[pl.ds(off, CH)])
                    return 0
                lax.fori_loop(0, nq, wout, 0)
            plsc.subcore_barrier()

    return cheb


def _make_mix(M, act):
    def mix(xr, tr, wr, br, outr):
        acc = jnp.zeros((BM, 32), F32)
        for t in range(2 * KTERMS):
            if t < 2:
                a = xr[t]
            else:
                a = tr[(t - 2) // 2, (t - 2) % 2]
            acc = acc + jnp.dot(a, wr[t * 16:(t + 1) * 16, :],
                                preferred_element_type=F32)
        acc = acc + br[...]
        if act:
            acc = jnp.where(acc >= 0, acc, 0.01 * acc)
        outr[0] = acc[:, :16]
        outr[1] = acc[:, 16:]

    return pl.pallas_call(
        mix,
        grid=(M // BM,),
        in_specs=[
            pl.BlockSpec((2, BM, 16), lambda i: (0, i, 0)),
            pl.BlockSpec((4, 2, BM, 16), lambda i: (0, 0, i, 0)),
            pl.BlockSpec((160, 32), lambda i: (0, 0)),
            pl.BlockSpec((1, 32), lambda i: (0, 0)),
        ],
        out_specs=pl.BlockSpec((2, BM, 16), lambda i: (0, i, 0)),
        out_shape=jax.ShapeDtypeStruct((2, M, 16), F32),
    )


def _mk_w(theta):
    o, i, _ = theta.shape
    th = jnp.pad(theta, ((0, 32 - o), (0, 32 - i), (0, 0)))
    return jnp.transpose(th, (2, 1, 0)).reshape(32 * KTERMS, 32)


def _mk_b(b):
    v = b.reshape(-1)
    return jnp.pad(v, (0, 32 - v.shape[0]))[None, :]


def _graph(idx, val, x, params):
    nnz = val.shape[0]
    m = x.shape[-1]
    mp = -(-m // BM) * BM
    nnzp = -(-nnz // (NS * G)) * (NS * G)
    rows = jnp.pad(idx[0], (0, nnzp - nnz))
    cols = jnp.pad(idx[1], (0, nnzp - nnz))
    vv = jnp.pad(val, (0, nnzp - nnz))
    xm = x.reshape(m)
    xin = jnp.zeros((2, mp, 16), F32).at[0, :m, 0].set(xm)

    cheb = _make_cheb(mp, nnzp)
    z = xin
    for layer, (th, b) in enumerate(params):
        t = cheb(z, rows, cols, vv)
        z = _make_mix(mp, layer < 2)(z, t, _mk_w(th), _mk_b(b))
    return z[0, :m, 0].reshape(1, 1, m)


def kernel(L0_idx, L0_val, L1_idx, L1_val, L2_idx, L2_val,
           D0, D1, D2, adD0, adD1, adD2, x0, x1, x2,
           t0_1, b0_1, t0_2, b0_2, t0_3, b0_3,
           t1_1, b1_1, t1_2, b1_2, t1_3, b1_3,
           t2_1, b2_1, t2_2, b2_2, t2_3, b2_3):
    o0 = _graph(L0_idx, L0_val, x0, [(t0_1, b0_1), (t0_2, b0_2), (t0_3, b0_3)])
    o1 = _graph(L1_idx, L1_val, x1, [(t1_1, b1_1), (t1_2, b1_2), (t1_3, b1_3)])
    o2 = _graph(L2_idx, L2_val, x2, [(t2_1, b2_1), (t2_2, b2_2), (t2_3, b2_3)])
    return (o0, o1, o2)


# speedup vs baseline: 7.7605x; 1.7742x over previous
import functools

import jax
import jax.numpy as jnp
from jax import lax
from jax.experimental import pallas as pl
from jax.experimental.pallas import tpu as pltpu
from jax.experimental.pallas import tpu_sc as plsc

F32 = jnp.float32
I32 = jnp.int32
NS = 16
G = 128
BK = 4
BM = 1024
KTERMS = 5

_DNUMS = lax.GatherDimensionNumbers(
    offset_dims=(), collapsed_slice_dims=(0,), start_index_map=(0,))


def _zchunk(rng, cap=512):
    best = 8
    for d in range(8, cap + 1, 8):
        if rng % d == 0:
            best = d
    return best


def _make_cheb(M, NNZp):
    rng = M // NS
    nper = NNZp // NS
    nch = nper // G
    nblk = nch // BK
    CH = _zchunk(rng)
    nq = rng // CH
    mesh = plsc.VectorSubcoreMesh(core_axis_name="c", subcore_axis_name="s")

    @functools.partial(
        pl.kernel,
        out_type=jax.ShapeDtypeStruct((KTERMS, 2, M, 16), F32),
        mesh=mesh,
        compiler_params=pltpu.CompilerParams(
            use_tc_tiling_on_sc=False, needs_layout_passes=False),
        scratch_types=[
            pltpu.VMEM_SHARED((M, 16), F32),
            pltpu.VMEM((CH, 16), F32),
            pltpu.VMEM((CH, 16), F32),
            pltpu.VMEM((2, BK, G, 16), F32),
            pltpu.VMEM((2, BK, 3, G), I32),
            pltpu.SemaphoreType.DMA,
            pltpu.SemaphoreType.DMA,
            pltpu.SemaphoreType.DMA,
            pltpu.SemaphoreType.DMA,
            pltpu.SemaphoreType.DMA,
            pltpu.SemaphoreType.DMA,
        ],
    )
    def cheb(xin, idxp, tout,
             Ysh, cbuf, pbuf, rbuf, ibuf, is0, is1, gs0, gs1, ss0, ss1):
        c = lax.axis_index("c")
        s = lax.axis_index("s")
        base = s * rng
        isem = (is0, is1)
        gsem = (gs0, gs1)
        ssem = (ss0, ss1)

        def fire_idx(pb, r):
            pltpu.async_copy(idxp.at[s, pl.ds(pb * BK, BK)],
                             ibuf.at[r], isem[r])

        def wait_idx(r):
            pltpu.make_async_copy(idxp.at[s, pl.ds(0, BK)],
                                  ibuf.at[r], isem[r]).wait()

        def fire_gathers(r, src):
            for q in range(BK):
                pltpu.async_copy(src.at[ibuf.at[r, q, 1]],
                                 rbuf.at[r, q], gsem[r])

        def drain_gathers(r, src):
            for q in range(BK):
                pltpu.make_async_copy(src.at[ibuf.at[r, q, 1]],
                                      rbuf.at[r, q], gsem[r]).wait()

        def drain_scatters(r):
            for q in range(BK):
                pltpu.make_async_copy(rbuf.at[r, q],
                                      Ysh.at[ibuf.at[r, q, 0]],
                                      ssem[r]).wait()

        def scale_scatter(r, q):
            def sbody(j, _):
                v16 = plsc.bitcast(ibuf[r, q, 2, pl.ds(j * 16, 16)], F32)
                for l in range(16):
                    b = lax.gather(
                        v16, jnp.full((16, 1), l, I32), _DNUMS, (1,),
                        mode=lax.GatherScatterMode.PROMISE_IN_BOUNDS)
                    jj = j * 16 + l
                    rbuf[r, q, jj, :] = rbuf[r, q, jj, :] * b
                return 0
            lax.fori_loop(0, G // 16, sbody, 0)
            pltpu.async_copy(rbuf.at[r, q], Ysh.at[ibuf.at[r, q, 0]],
                             ssem[r], add=True)

        def block_body(p, r, src, skip_other=False, do_next=True):
            ro = 1 - r
            drain_gathers(r, src)
            scale_scatter(r, 0)
            scale_scatter(r, 1)
            if not skip_other:
                drain_scatters(ro)
            if do_next:
                fire_idx(p + 1, ro)
            scale_scatter(r, 2)
            scale_scatter(r, 3)
            if do_next:
                wait_idx(ro)
                fire_gathers(ro, src)

        def zero_accum():
            def zfill(i, _):
                cbuf[i, :] = jnp.zeros((16,), F32)
                return 0
            lax.fori_loop(0, CH, zfill, 0)

            def zrow(i, _):
                pltpu.sync_copy(cbuf, Ysh.at[pl.ds(base + i * CH, CH)])
                return 0
            lax.fori_loop(0, nq, zrow, 0)
            plsc.subcore_barrier()

        def scatter_pass(src):
            fire_idx(0, 0)
            wait_idx(0)
            fire_gathers(0, src)
            block_body(0, 0, src, skip_other=True)

            def pair(j, _):
                block_body(1 + 2 * j, 1, src)
                block_body(2 + 2 * j, 0, src)
                return 0
            lax.fori_loop(0, (nblk - 2) // 2, pair, 0)
            block_body(nblk - 1, 1, src, do_next=False)
            drain_scatters(1)
            plsc.subcore_barrier()

        zero_accum()
        scatter_pass(xin.at[c])

        def wout1(i, _):
            off = base + i * CH
            pltpu.sync_copy(Ysh.at[pl.ds(off, CH)], cbuf)
            pltpu.sync_copy(cbuf, tout.at[1, c, pl.ds(off, CH)])
            pltpu.sync_copy(xin.at[c, pl.ds(off, CH)], pbuf)
            pltpu.sync_copy(pbuf, tout.at[0, c, pl.ds(off, CH)])
            return 0
        lax.fori_loop(0, nq, wout1, 0)
        plsc.subcore_barrier()

        def step(kk, _):
            zero_accum()
            scatter_pass(tout.at[kk - 1, c])

            def wout(i, _):
                off = base + i * CH
                pltpu.sync_copy(Ysh.at[pl.ds(off, CH)], cbuf)
                pltpu.sync_copy(tout.at[kk - 2, c, pl.ds(off, CH)], pbuf)

                def comb(i2, _):
                    cbuf[i2, :] = 2.0 * cbuf[i2, :] - pbuf[i2, :]
                    return 0
                lax.fori_loop(0, CH, comb, 0)
                pltpu.sync_copy(cbuf, tout.at[kk, c, pl.ds(off, CH)])
                return 0
            lax.fori_loop(0, nq, wout, 0)
            plsc.subcore_barrier()
            return 0
        lax.fori_loop(2, KTERMS, step, 0)

    return cheb


def _make_mix(M, act):
    def mix(tr, wr, br, outr):
        acc = jnp.zeros((BM, 32), F32)
        for t in range(2 * KTERMS):
            a = tr[t // 2, t % 2]
            acc = acc + jnp.dot(a, wr[t * 16:(t + 1) * 16, :],
                                preferred_element_type=F32)
        acc = acc + br[...]
        if act:
            acc = jnp.where(acc >= 0, acc, 0.01 * acc)
        outr[0] = acc[:, :16]
        outr[1] = acc[:, 16:]

    return pl.pallas_call(
        mix,
        grid=(M // BM,),
        in_specs=[
            pl.BlockSpec((KTERMS, 2, BM, 16), lambda i: (0, 0, i, 0)),
            pl.BlockSpec((160, 32), lambda i: (0, 0)),
            pl.BlockSpec((1, 32), lambda i: (0, 0)),
        ],
        out_specs=pl.BlockSpec((2, BM, 16), lambda i: (0, i, 0)),
        out_shape=jax.ShapeDtypeStruct((2, M, 16), F32),
    )


def _mk_w(theta):
    o, i, _ = theta.shape
    th = jnp.pad(theta, ((0, 32 - o), (0, 32 - i), (0, 0)))
    return jnp.transpose(th, (2, 1, 0)).reshape(32 * KTERMS, 32)


def _mk_b(b):
    v = b.reshape(-1)
    return jnp.pad(v, (0, 32 - v.shape[0]))[None, :]


def _graph(idx, val, x, params):
    nnz = val.shape[0]
    m = x.shape[-1]
    mp = -(-m // BM) * BM
    unit = NS * G * 2 * BK
    nnzp = -(-nnz // unit) * unit
    rows = jnp.pad(idx[0], (0, nnzp - nnz))
    cols = jnp.pad(idx[1], (0, nnzp - nnz))
    vv = jnp.pad(val, (0, nnzp - nnz))
    nch = nnzp // (NS * G)
    packed = jnp.stack(
        [rows.reshape(NS, nch, G),
         cols.reshape(NS, nch, G),
         lax.bitcast_convert_type(vv, I32).reshape(NS, nch, G)], axis=2)
    xm = x.reshape(m)
    xin = jnp.zeros((2, mp, 16), F32).at[0, :m, 0].set(xm)

    cheb = _make_cheb(mp, nnzp)
    z = xin
    for layer, (th, b) in enumerate(params):
        t = cheb(z, packed)
        z = _make_mix(mp, layer < 2)(t, _mk_w(th), _mk_b(b))
    return z[0, :m, 0].reshape(1, 1, m)


def kernel(L0_idx, L0_val, L1_idx, L1_val, L2_idx, L2_val,
           D0, D1, D2, adD0, adD1, adD2, x0, x1, x2,
           t0_1, b0_1, t0_2, b0_2, t0_3, b0_3,
           t1_1, b1_1, t1_2, b1_2, t1_3, b1_3,
           t2_1, b2_1, t2_2, b2_2, t2_3, b2_3):
    o0 = _graph(L0_idx, L0_val, x0, [(t0_1, b0_1), (t0_2, b0_2), (t0_3, b0_3)])
    o1 = _graph(L1_idx, L1_val, x1, [(t1_1, b1_1), (t1_2, b1_2), (t1_3, b1_3)])
    o2 = _graph(L2_idx, L2_val, x2, [(t2_1, b2_1), (t2_2, b2_2), (t2_3, b2_3)])
    return (o0, o1, o2)


# speedup vs baseline: 12.2327x; 1.5763x over previous
import functools

import jax
import jax.numpy as jnp
from jax import lax
from jax.experimental import pallas as pl
from jax.experimental.pallas import tpu as pltpu
from jax.experimental.pallas import tpu_sc as plsc

F32 = jnp.float32
I32 = jnp.int32
NS = 16
G = 128
BK = 2
BM = 1024
KTERMS = 5

_DNUMS = lax.GatherDimensionNumbers(
    offset_dims=(), collapsed_slice_dims=(0,), start_index_map=(0,))


def _zchunk(rng, cap=256):
    best = 8
    for d in range(8, cap + 1, 8):
        if rng % d == 0:
            best = d
    return best


def _make_cheb(M, NNZp):
    rng = M // NS
    nper = NNZp // NS
    nch = nper // G
    nblk = nch // BK
    CH = _zchunk(rng)
    nq = rng // CH
    mesh = plsc.VectorSubcoreMesh(core_axis_name="c", subcore_axis_name="s")

    @functools.partial(
        pl.kernel,
        out_type=jax.ShapeDtypeStruct((KTERMS, 2, M, 16), F32),
        mesh=mesh,
        compiler_params=pltpu.CompilerParams(
            use_tc_tiling_on_sc=False, needs_layout_passes=False),
        scratch_types=[
            pltpu.VMEM_SHARED((M, 16), F32),
            pltpu.VMEM_SHARED((M, 16), F32),
            pltpu.VMEM((CH, 16), F32),
            pltpu.VMEM((CH, 16), F32),
            pltpu.VMEM((CH, 16), F32),
            pltpu.VMEM((2, BK, G, 16), F32),
            pltpu.VMEM((2, BK, 3, G), I32),
            pltpu.SemaphoreType.DMA,
            pltpu.SemaphoreType.DMA,
            pltpu.SemaphoreType.DMA,
            pltpu.SemaphoreType.DMA,
            pltpu.SemaphoreType.DMA,
            pltpu.SemaphoreType.DMA,
        ],
    )
    def cheb(xin, idxp, tout,
             Ysh, Tsh, cbuf, pbuf, zbuf, rbuf, ibuf,
             is0, is1, gs0, gs1, ss0, ss1):
        c = lax.axis_index("c")
        s = lax.axis_index("s")
        base = s * rng
        isem = (is0, is1)
        gsem = (gs0, gs1)
        ssem = (ss0, ss1)

        def fire_idx(pb, r):
            pltpu.async_copy(idxp.at[s, pl.ds(pb * BK, BK)],
                             ibuf.at[r], isem[r])

        def wait_idx(r):
            pltpu.make_async_copy(idxp.at[s, pl.ds(0, BK)],
                                  ibuf.at[r], isem[r]).wait()

        def fire_gathers(r):
            for q in range(BK):
                pltpu.async_copy(Tsh.at[ibuf.at[r, q, 1]],
                                 rbuf.at[r, q], gsem[r])

        def drain_gathers(r):
            for q in range(BK):
                pltpu.make_async_copy(Tsh.at[ibuf.at[r, q, 1]],
                                      rbuf.at[r, q], gsem[r]).wait()

        def drain_scatters(r):
            for q in range(BK):
                pltpu.make_async_copy(rbuf.at[r, q],
                                      Ysh.at[ibuf.at[r, q, 0]],
                                      ssem[r]).wait()

        def scale_scatter(r, q):
            def sbody(j, _):
                v16 = plsc.bitcast(ibuf[r, q, 2, pl.ds(j * 16, 16)], F32)
                for l in range(16):
                    b = lax.gather(
                        v16, jnp.full((16, 1), l, I32), _DNUMS, (1,),
                        mode=lax.GatherScatterMode.PROMISE_IN_BOUNDS)
                    jj = j * 16 + l
                    rbuf[r, q, jj, :] = rbuf[r, q, jj, :] * b
                return 0
            lax.fori_loop(0, G // 16, sbody, 0)
            pltpu.async_copy(rbuf.at[r, q], Ysh.at[ibuf.at[r, q, 0]],
                             ssem[r], add=True)

        def block_body(p, r, skip_other=False, do_next=True):
            ro = 1 - r
            drain_gathers(r)
            scale_scatter(r, 0)
            if not skip_other:
                drain_scatters(ro)
            if do_next:
                fire_idx(p + 1, ro)
            scale_scatter(r, 1)
            if do_next:
                wait_idx(ro)
                fire_gathers(ro)

        def zero_accum():
            def zrow(i, _):
                pltpu.sync_copy(zbuf, Ysh.at[pl.ds(base + i * CH, CH)])
                return 0
            lax.fori_loop(0, nq, zrow, 0)
            plsc.subcore_barrier()

        def scatter_pass():
            fire_idx(0, 0)
            wait_idx(0)
            fire_gathers(0)
            block_body(0, 0, skip_other=True)

            def pair(j, _):
                block_body(1 + 2 * j, 1)
                block_body(2 + 2 * j, 0)
                return 0
            lax.fori_loop(0, (nblk - 2) // 2, pair, 0)
            block_body(nblk - 1, 1, do_next=False)
            drain_scatters(1)
            plsc.subcore_barrier()

        def zfill(i, _):
            zbuf[i, :] = jnp.zeros((16,), F32)
            return 0
        lax.fori_loop(0, CH, zfill, 0)

        def pre(i, _):
            off = base + i * CH
            pltpu.sync_copy(xin.at[c, pl.ds(off, CH)], pbuf)
            pltpu.sync_copy(pbuf, Tsh.at[pl.ds(off, CH)])
            pltpu.sync_copy(pbuf, tout.at[0, c, pl.ds(off, CH)])
            return 0
        lax.fori_loop(0, nq, pre, 0)

        zero_accum()
        scatter_pass()

        def wout1(i, _):
            off = base + i * CH
            pltpu.sync_copy(Ysh.at[pl.ds(off, CH)], cbuf)
            pltpu.sync_copy(zbuf, Ysh.at[pl.ds(off, CH)])
            pltpu.sync_copy(cbuf, tout.at[1, c, pl.ds(off, CH)])
            pltpu.sync_copy(cbuf, Tsh.at[pl.ds(off, CH)])
            return 0
        lax.fori_loop(0, nq, wout1, 0)
        plsc.subcore_barrier()

        def step(kk, _):
            scatter_pass()

            def wout(i, _):
                off = base + i * CH
                pltpu.sync_copy(Ysh.at[pl.ds(off, CH)], cbuf)
                pltpu.sync_copy(zbuf, Ysh.at[pl.ds(off, CH)])
                pltpu.sync_copy(tout.at[kk - 2, c, pl.ds(off, CH)], pbuf)

                def comb(i2, _):
                    cbuf[i2, :] = 2.0 * cbuf[i2, :] - pbuf[i2, :]
                    return 0
                lax.fori_loop(0, CH, comb, 0)
                pltpu.sync_copy(cbuf, tout.at[kk, c, pl.ds(off, CH)])
                pltpu.sync_copy(cbuf, Tsh.at[pl.ds(off, CH)])
                return 0
            lax.fori_loop(0, nq, wout, 0)
            plsc.subcore_barrier()
            return 0
        lax.fori_loop(2, KTERMS, step, 0)

    return cheb


def _make_mix(M, act):
    def mix(tr, wr, br, outr):
        acc = jnp.zeros((BM, 32), F32)
        for t in range(2 * KTERMS):
            a = tr[t // 2, t % 2]
            acc = acc + jnp.dot(a, wr[t * 16:(t + 1) * 16, :],
                                preferred_element_type=F32)
        acc = acc + br[...]
        if act:
            acc = jnp.where(acc >= 0, acc, 0.01 * acc)
        outr[0] = acc[:, :16]
        outr[1] = acc[:, 16:]

    return pl.pallas_call(
        mix,
        grid=(M // BM,),
        in_specs=[
            pl.BlockSpec((KTERMS, 2, BM, 16), lambda i: (0, 0, i, 0)),
            pl.BlockSpec((160, 32), lambda i: (0, 0)),
            pl.BlockSpec((1, 32), lambda i: (0, 0)),
        ],
        out_specs=pl.BlockSpec((2, BM, 16), lambda i: (0, i, 0)),
        out_shape=jax.ShapeDtypeStruct((2, M, 16), F32),
    )


def _mk_w(theta):
    o, i, _ = theta.shape
    th = jnp.pad(theta, ((0, 32 - o), (0, 32 - i), (0, 0)))
    return jnp.transpose(th, (2, 1, 0)).reshape(32 * KTERMS, 32)


def _mk_b(b):
    v = b.reshape(-1)
    return jnp.pad(v, (0, 32 - v.shape[0]))[None, :]


def _graph(idx, val, x, params):
    nnz = val.shape[0]
    m = x.shape[-1]
    mp = -(-m // BM) * BM
    unit = NS * G * 2 * BK
    nnzp = -(-nnz // unit) * unit
    rows = jnp.pad(idx[0], (0, nnzp - nnz))
    cols = jnp.pad(idx[1], (0, nnzp - nnz))
    vv = jnp.pad(val, (0, nnzp - nnz))
    nch = nnzp // (NS * G)
    packed = jnp.stack(
        [rows.reshape(NS, nch, G),
         cols.reshape(NS, nch, G),
         lax.bitcast_convert_type(vv, I32).reshape(NS, nch, G)], axis=2)
    xm = x.reshape(m)
    xin = jnp.zeros((2, mp, 16), F32).at[0, :m, 0].set(xm)

    cheb = _make_cheb(mp, nnzp)
    z = xin
    for layer, (th, b) in enumerate(params):
        t = cheb(z, packed)
        z = _make_mix(mp, layer < 2)(t, _mk_w(th), _mk_b(b))
    return z[0, :m, 0].reshape(1, 1, m)


def kernel(L0_idx, L0_val, L1_idx, L1_val, L2_idx, L2_val,
           D0, D1, D2, adD0, adD1, adD2, x0, x1, x2,
           t0_1, b0_1, t0_2, b0_2, t0_3, b0_3,
           t1_1, b1_1, t1_2, b1_2, t1_3, b1_3,
           t2_1, b2_1, t2_2, b2_2, t2_3, b2_3):
    o0 = _graph(L0_idx, L0_val, x0, [(t0_1, b0_1), (t0_2, b0_2), (t0_3, b0_3)])
    o1 = _graph(L1_idx, L1_val, x1, [(t1_1, b1_1), (t1_2, b1_2), (t1_3, b1_3)])
    o2 = _graph(L2_idx, L2_val, x2, [(t2_1, b2_1), (t2_2, b2_2), (t2_3, b2_3)])
    return (o0, o1, o2)


# speedup vs baseline: 15.5911x; 1.2745x over previous
import functools

import jax
import jax.numpy as jnp
from jax import lax
from jax.experimental import pallas as pl
from jax.experimental.pallas import tpu as pltpu
from jax.experimental.pallas import tpu_sc as plsc

F32 = jnp.float32
I32 = jnp.int32
NS = 16
G = 128
BK = 2
BM = 1024
KTERMS = 5

_DNUMS = lax.GatherDimensionNumbers(
    offset_dims=(), collapsed_slice_dims=(0,), start_index_map=(0,))


def _zchunk(rng, cap=128):
    best = 8
    for d in range(8, cap + 1, 8):
        if rng % d == 0:
            best = d
    return best


def _make_cheb(M, NNZp):
    rng = M // NS
    nper = NNZp // NS
    nch = nper // G
    nblk = nch // BK
    CH = _zchunk(rng)
    nq = rng // CH
    mesh = plsc.VectorSubcoreMesh(core_axis_name="c", subcore_axis_name="s")

    @functools.partial(
        pl.kernel,
        out_type=jax.ShapeDtypeStruct((KTERMS, 2, M, 16), F32),
        mesh=mesh,
        compiler_params=pltpu.CompilerParams(
            use_tc_tiling_on_sc=False, needs_layout_passes=False),
        scratch_types=[
            pltpu.VMEM_SHARED((M, 16), F32),
            pltpu.VMEM_SHARED((M, 16), F32),
            pltpu.VMEM((2, CH, 16), F32),
            pltpu.VMEM((2, CH, 16), F32),
            pltpu.VMEM((CH, 16), F32),
            pltpu.VMEM((2, BK, G, 16), F32),
            pltpu.VMEM((4, BK, 3, G), I32),
            pltpu.SemaphoreType.DMA,
            pltpu.SemaphoreType.DMA,
            pltpu.SemaphoreType.DMA,
            pltpu.SemaphoreType.DMA,
            pltpu.SemaphoreType.DMA,
            pltpu.SemaphoreType.DMA,
            pltpu.SemaphoreType.DMA,
            pltpu.SemaphoreType.DMA,
            pltpu.SemaphoreType.DMA,
            pltpu.SemaphoreType.DMA,
            pltpu.SemaphoreType.DMA,
            pltpu.SemaphoreType.DMA,
        ],
    )
    def cheb(xin, idxp, tout,
             Ysh, Tsh, cbuf, pbuf, zbuf, rbuf, ibuf,
             is0, is1, is2, is3, gs0, gs1, ss0, ss1, ps0, ps1, ws0, ws1):
        c = lax.axis_index("c")
        s = lax.axis_index("s")
        base = s * rng
        isem = (is0, is1, is2, is3)
        gsem = (gs0, gs1)
        ssem = (ss0, ss1)
        psem = (ps0, ps1)
        wsem = (ws0, ws1)

        def fire_idx(pb, t):
            pltpu.async_copy(idxp.at[s, pl.ds(pb * BK, BK)],
                             ibuf.at[t], isem[t])

        def wait_idx(t):
            pltpu.make_async_copy(idxp.at[s, pl.ds(0, BK)],
                                  ibuf.at[t], isem[t]).wait()

        def fire_gathers(t, r):
            for q in range(BK):
                pltpu.async_copy(Tsh.at[ibuf.at[t, q, 1]],
                                 rbuf.at[r, q], gsem[r])

        def drain_gathers(t, r):
            for q in range(BK):
                pltpu.make_async_copy(Tsh.at[ibuf.at[t, q, 1]],
                                      rbuf.at[r, q], gsem[r]).wait()

        def drain_scatters(t, r):
            for q in range(BK):
                pltpu.make_async_copy(rbuf.at[r, q],
                                      Ysh.at[ibuf.at[t, q, 0]],
                                      ssem[r]).wait()

        def scale_scatter(t, r, q):
            def sbody(j, _):
                v16 = plsc.bitcast(ibuf[t, q, 2, pl.ds(j * 16, 16)], F32)
                for l in range(16):
                    b = lax.gather(
                        v16, jnp.full((16, 1), l, I32), _DNUMS, (1,),
                        mode=lax.GatherScatterMode.PROMISE_IN_BOUNDS)
                    jj = j * 16 + l
                    rbuf[r, q, jj, :] = rbuf[r, q, jj, :] * b
                return 0
            lax.fori_loop(0, G // 16, sbody, 0)
            pltpu.async_copy(rbuf.at[r, q], Ysh.at[ibuf.at[t, q, 0]],
                             ssem[r], add=True)

        def block_body(p, t, r, first=False, fire3=True, nxt=True):
            drain_gathers(t, r)
            scale_scatter(t, r, 0)
            if not first:
                drain_scatters((t - 1) % 4, 1 - r)
            if fire3:
                fire_idx(p + 3, (t + 3) % 4)
            scale_scatter(t, r, 1)
            if nxt:
                wait_idx((t + 1) % 4)
                fire_gathers((t + 1) % 4, 1 - r)

        def zero_accum():
            def zrow(i, _):
                pltpu.sync_copy(zbuf, Ysh.at[pl.ds(base + i * CH, CH)])
                return 0
            lax.fori_loop(0, nq, zrow, 0)
            plsc.subcore_barrier()

        def scatter_pass():
            for t in range(3):
                fire_idx(t, t)
            wait_idx(0)
            fire_gathers(0, 0)
            block_body(0, 0, 0, first=True)
            block_body(1, 1, 1)
            block_body(2, 2, 0)
            block_body(3, 3, 1)

            def quad(j, _):
                p = 4 * j
                block_body(p, 0, 0)
                block_body(p + 1, 1, 1)
                block_body(p + 2, 2, 0)
                block_body(p + 3, 3, 1)
                return 0
            lax.fori_loop(1, nblk // 4 - 1, quad, 0)
            p = nblk - 4
            block_body(p, 0, 0)
            block_body(p + 1, 1, 1, fire3=False)
            block_body(p + 2, 2, 0, fire3=False)
            block_body(p + 3, 3, 1, fire3=False, nxt=False)
            drain_scatters(3, 1)
            plsc.subcore_barrier()

        def zfill(i, _):
            zbuf[i, :] = jnp.zeros((16,), F32)
            return 0
        lax.fori_loop(0, CH, zfill, 0)

        def pre(i, _):
            off = base + i * CH
            pltpu.sync_copy(xin.at[c, pl.ds(off, CH)], pbuf.at[0])
            pltpu.sync_copy(pbuf.at[0], Tsh.at[pl.ds(off, CH)])
            pltpu.sync_copy(pbuf.at[0], tout.at[0, c, pl.ds(off, CH)])
            return 0
        lax.fori_loop(0, nq, pre, 0)

        def fire_w(k, i, par):
            pltpu.async_copy(cbuf.at[par],
                             tout.at[k, c, pl.ds(base + i * CH, CH)],
                             wsem[par])

        def wait_w(k, i, par):
            pltpu.make_async_copy(cbuf.at[par],
                                  tout.at[k, c, pl.ds(base + i * CH, CH)],
                                  wsem[par]).wait()

        zero_accum()
        scatter_pass()

        for i in range(nq):
            par = i & 1
            off = base + i * CH
            if i >= 2:
                wait_w(1, i - 2, par)
            pltpu.sync_copy(Ysh.at[pl.ds(off, CH)], cbuf.at[par])
            pltpu.sync_copy(zbuf, Ysh.at[pl.ds(off, CH)])
            fire_w(1, i, par)
            pltpu.sync_copy(cbuf.at[par], Tsh.at[pl.ds(off, CH)])
        if nq >= 2:
            wait_w(1, nq - 2, (nq - 2) & 1)
        wait_w(1, nq - 1, (nq - 1) & 1)
        plsc.subcore_barrier()

        def step(kk, _):
            scatter_pass()

            def fire_p(i, par):
                pltpu.async_copy(tout.at[kk - 2, c, pl.ds(base + i * CH, CH)],
                                 pbuf.at[par], psem[par])

            def wait_p(i, par):
                pltpu.make_async_copy(
                    tout.at[kk - 2, c, pl.ds(base + i * CH, CH)],
                    pbuf.at[par], psem[par]).wait()

            fire_p(0, 0)
            for i in range(nq):
                par = i & 1
                off = base + i * CH
                if i + 1 < nq:
                    fire_p(i + 1, 1 - par)
                if i >= 2:
                    wait_w(kk, i - 2, par)
                pltpu.sync_copy(Ysh.at[pl.ds(off, CH)], cbuf.at[par])
                pltpu.sync_copy(zbuf, Ysh.at[pl.ds(off, CH)])
                wait_p(i, par)

                def comb(i2, _):
                    cbuf[par, i2, :] = 2.0 * cbuf[par, i2, :] - pbuf[par, i2, :]
                    return 0
                lax.fori_loop(0, CH, comb, 0)
                fire_w(kk, i, par)
                pltpu.sync_copy(cbuf.at[par], Tsh.at[pl.ds(off, CH)])
            if nq >= 2:
                wait_w(kk, nq - 2, (nq - 2) & 1)
            wait_w(kk, nq - 1, (nq - 1) & 1)
            plsc.subcore_barrier()
            return 0
        lax.fori_loop(2, KTERMS, step, 0)

    return cheb


def _make_mix(M, act):
    def mix(tr, wr, br, outr):
        acc = jnp.zeros((BM, 32), F32)
        for t in range(2 * KTERMS):
            a = tr[t // 2, t % 2]
            acc = acc + jnp.dot(a, wr[t * 16:(t + 1) * 16, :],
                                preferred_element_type=F32)
        acc = acc + br[...]
        if act:
            acc = jnp.where(acc >= 0, acc, 0.01 * acc)
        outr[0] = acc[:, :16]
        outr[1] = acc[:, 16:]

    return pl.pallas_call(
        mix,
        grid=(M // BM,),
        in_specs=[
            pl.BlockSpec((KTERMS, 2, BM, 16), lambda i: (0, 0, i, 0)),
            pl.BlockSpec((160, 32), lambda i: (0, 0)),
            pl.BlockSpec((1, 32), lambda i: (0, 0)),
        ],
        out_specs=pl.BlockSpec((2, BM, 16), lambda i: (0, i, 0)),
        out_shape=jax.ShapeDtypeStruct((2, M, 16), F32),
    )


def _mk_w(theta):
    o, i, _ = theta.shape
    th = jnp.pad(theta, ((0, 32 - o), (0, 32 - i), (0, 0)))
    return jnp.transpose(th, (2, 1, 0)).reshape(32 * KTERMS, 32)


def _mk_b(b):
    v = b.reshape(-1)
    return jnp.pad(v, (0, 32 - v.shape[0]))[None, :]


def _graph(idx, val, x, params):
    nnz = val.shape[0]
    m = x.shape[-1]
    mp = -(-m // BM) * BM
    unit = NS * G * 8 * BK
    nnzp = -(-nnz // unit) * unit
    rows = jnp.pad(idx[0], (0, nnzp - nnz))
    cols = jnp.pad(idx[1], (0, nnzp - nnz))
    vv = jnp.pad(val, (0, nnzp - nnz))
    nch = nnzp // (NS * G)
    packed = jnp.stack(
        [rows.reshape(NS, nch, G),
         cols.reshape(NS, nch, G),
         lax.bitcast_convert_type(vv, I32).reshape(NS, nch, G)], axis=2)
    xm = x.reshape(m)
    xin = jnp.zeros((2, mp, 16), F32).at[0, :m, 0].set(xm)

    cheb = _make_cheb(mp, nnzp)
    z = xin
    for layer, (th, b) in enumerate(params):
        t = cheb(z, packed)
        z = _make_mix(mp, layer < 2)(t, _mk_w(th), _mk_b(b))
    return z[0, :m, 0].reshape(1, 1, m)


def kernel(L0_idx, L0_val, L1_idx, L1_val, L2_idx, L2_val,
           D0, D1, D2, adD0, adD1, adD2, x0, x1, x2,
           t0_1, b0_1, t0_2, b0_2, t0_3, b0_3,
           t1_1, b1_1, t1_2, b1_2, t1_3, b1_3,
           t2_1, b2_1, t2_2, b2_2, t2_3, b2_3):
    o0 = _graph(L0_idx, L0_val, x0, [(t0_1, b0_1), (t0_2, b0_2), (t0_3, b0_3)])
    o1 = _graph(L1_idx, L1_val, x1, [(t1_1, b1_1), (t1_2, b1_2), (t1_3, b1_3)])
    o2 = _graph(L2_idx, L2_val, x2, [(t2_1, b2_1), (t2_2, b2_2), (t2_3, b2_3)])
    return (o0, o1, o2)
